# baseline (speedup 1.0000x reference)
import functools

import jax
import jax.numpy as jnp
from jax import lax
from jax.experimental import pallas as pl
from jax.experimental.pallas import tpu as pltpu
from jax.experimental.pallas import tpu_sc as plsc

DIM = 64
SCALE = 8.0
NC, NS, LANES = 2, 16, 16
NW = NC * NS
NBUF = 4
XBUF = 2
NT = 31
NRC = NT * 8
P1CH = 126

_params = pltpu.CompilerParams(
    use_tc_tiling_on_sc=True, needs_layout_passes=False
)
_mesh = lambda: plsc.VectorSubcoreMesh(core_axis_name="c", subcore_axis_name="s")


def _depad(table):
    V = table.shape[0]

    @functools.partial(
        pl.kernel,
        mesh=_mesh(),
        compiler_params=_params,
        out_type=jax.ShapeDtypeStruct((V, 2 * DIM), jnp.float32),
        scratch_types=[
            pltpu.VMEM((2, NRC, DIM), jnp.float32),
            pltpu.VMEM((2, NRC, 2 * DIM), jnp.float32),
        ]
        + [pltpu.SemaphoreType.DMA] * 4,
    )
    def depad_kernel(tab_hbm, out_hbm, vin, vout, *sems):
        isem = sems[:2]
        osem = sems[2:]
        wid = lax.axis_index("s") * NC + lax.axis_index("c")
        ntile = V // 8
        t_lo = ntile * wid // NW
        nt_w = ntile * (wid + 1) // NW - t_lo
        r_lo = t_lo * 8

        def in_copy(c, b):
            return pltpu.make_async_copy(
                tab_hbm.at[pl.ds(r_lo + c * NRC, NRC)], vin.at[b], isem[b]
            )

        def out_copy(c, b):
            return pltpu.make_async_copy(
                vout.at[b],
                out_hbm.at[pl.ds(r_lo + c * NRC, NRC)],
                osem[b],
            )

        def pack(b, nr):
            @plsc.parallel_loop(0, nr, 1, unroll=4)
            def _(r):
                for h in range(DIM // LANES):
                    sl = pl.ds(LANES * h, LANES)
                    vout[b, r, sl] = vin[b, r, sl]

        in_copy(0, 0).start()
        in_copy(1, 1).start()

        @pl.loop(0, P1CH // 2)
        def outer(t):
            for b in range(2):
                c = 2 * t + b
                in_copy(c, b).wait()

                @pl.when(t > 0)
                def _():
                    out_copy(c - 2, b).wait()

                pack(b, NRC)
                out_copy(c, b).start()

                @pl.when(t < P1CH // 2 - 1)
                def _():
                    in_copy(c + 2, b).start()

        out_copy(P1CH - 2, 0).wait()
        out_copy(P1CH - 1, 1).wait()

        @pl.when(nt_w > P1CH * NT)
        def _tail():
            rt = r_lo + P1CH * NRC
            pltpu.make_async_copy(
                tab_hbm.at[pl.ds(rt, 8)], vin.at[0].at[pl.ds(0, 8)], isem[0]
            ).start()
            pltpu.make_async_copy(
                tab_hbm.at[pl.ds(rt, 8)], vin.at[0].at[pl.ds(0, 8)], isem[0]
            ).wait()
            pack(0, 8)
            pltpu.make_async_copy(
                vout.at[0].at[pl.ds(0, 8)],
                out_hbm.at[pl.ds(rt, 8)],
                osem[0],
            ).start()
            pltpu.make_async_copy(
                vout.at[0].at[pl.ds(0, 8)],
                out_hbm.at[pl.ds(rt, 8)],
                osem[0],
            ).wait()

    return depad_kernel(table)


def kernel(tokens, table):
    B, L = tokens.shape
    BW = B // NW
    CT = DIM // 8
    nt = L // NBUF

    padded = _depad(table)

    idx3 = jnp.transpose(
        tokens.astype(jnp.int32).reshape(NW, BW, L), (0, 2, 1)
    )

    @functools.partial(
        pl.kernel,
        mesh=_mesh(),
        compiler_params=_params,
        out_type=jax.ShapeDtypeStruct((L, CT, NW, 8, LANES * 8), jnp.float32),
        scratch_types=[
            pltpu.VMEM((L, BW), jnp.int32),
            pltpu.VMEM((NBUF, BW, 2 * DIM), jnp.float32),
            pltpu.VMEM((XBUF, DIM, BW), jnp.float32),
        ]
        + [pltpu.SemaphoreType.DMA] * (NBUF + XBUF),
    )
    def emb_kernel(idx_hbm, tab_hbm, out_hbm, tok_v, raw_v, xout_v, *sems):
        gsem = sems[:NBUF]
        osem = sems[NBUF:]
        wid = lax.axis_index("s") * NC + lax.axis_index("c")
        pltpu.sync_copy(idx_hbm.at[wid], tok_v)
        lane_iota = lax.iota(jnp.int32, LANES)

        def gather(c, b):
            return pltpu.make_async_copy(
                tab_hbm.at[tok_v.at[c]], raw_v.at[b], gsem[b]
            )

        def slab_stores(c, xb):
            return [
                pltpu.make_async_copy(
                    xout_v.at[xb].at[pl.ds(ct * 8, 8)],
                    out_hbm.at[c, ct, wid],
                    osem[xb],
                )
                for ct in range(CT)
            ]

        gather(0, 0).start()
        gather(1, 1).start()

        @pl.loop(0, nt)
        def outer(t):
            for b in range(NBUF):
                c = t * NBUF + b
                xb = b % XBUF
                if b < 2:
                    gather(c + 2, (b + 2) % NBUF).start()
                else:
                    @pl.when(t < nt - 1)
                    def _():
                        gather(c + 2, (b + 2) % NBUF).start()

                gather(c, b).wait()

                if b < 2:
                    @pl.when(t > 0)
                    def _():
                        for cp in slab_stores(c - 2, xb):
                            cp.wait()
                else:
                    for cp in slab_stores(c - 2, xb):
                        cp.wait()

                @plsc.parallel_loop(0, LANES * 32, 1, unroll=8)
                def transpose_scale(q):
                    s = q & 15
                    blk = q >> 4
                    c0 = (blk >> 3) * LANES
                    r0 = (blk & 7) * LANES
                    rot = (lane_iota + s) & 15
                    rows = lane_iota + r0
                    cols = rot + c0
                    v = plsc.load_gather(raw_v.at[b], [rows, cols])
                    plsc.store_scatter(
                        xout_v.at[xb], [cols, rows], v * SCALE
                    )

                for cp in slab_stores(c, xb):
                    cp.start()

        for cc, xb in ((L - 2, 0), (L - 1, 1)):
            for cp in slab_stores(cc, xb):
                cp.wait()

    out5 = emb_kernel(idx3, padded)
    return jnp.transpose(out5, (2, 4, 0, 1, 3)).reshape(B, L, DIM)


# speedup vs baseline: 1.6909x; 1.6909x over previous
import functools

import jax
import jax.numpy as jnp
from jax import lax
from jax.experimental import pallas as pl
from jax.experimental.pallas import tpu as pltpu
from jax.experimental.pallas import tpu_sc as plsc

DIM = 64
SCALE = 8.0
NC, NS, LANES = 2, 16, 16
NW = NC * NS
NBUF = 4
XBUF = 2
VB = 128
CHB = 2

_params = pltpu.CompilerParams(
    use_tc_tiling_on_sc=True, needs_layout_passes=False
)


def _mesh():
    return plsc.VectorSubcoreMesh(core_axis_name="c", subcore_axis_name="s")


def _transpose_table(table):
    V = table.shape[0]
    nfull = V // VB
    vtail = nfull * VB
    ntail = V - vtail
    tabT = jnp.transpose(table)
    tail = table[vtail:]

    @functools.partial(
        pl.kernel,
        mesh=_mesh(),
        compiler_params=_params,
        out_type=jax.ShapeDtypeStruct((V, 2 * DIM), jnp.float32),
        scratch_types=[
            pltpu.VMEM((2, DIM, CHB * VB), jnp.float32),
            pltpu.VMEM((2, CHB * VB, 2 * DIM), jnp.float32),
            pltpu.VMEM((ntail, DIM), jnp.float32),
        ]
        + [pltpu.SemaphoreType.DMA] * 4,
    )
    def transpose_kernel(tabT_hbm, tail_hbm, out_hbm, vin, vout, vt, *sems):
        isem = sems[:2]
        osem = sems[2:]
        wid = lax.axis_index("s") * NC + lax.axis_index("c")
        b_lo = nfull * wid // NW
        nblk = nfull * (wid + 1) // NW - b_lo
        nch = 122
        lane_iota = lax.iota(jnp.int32, LANES)

        def in_copy(ch, b):
            return pltpu.make_async_copy(
                tabT_hbm.at[:, pl.ds((b_lo + CHB * ch) * VB, CHB * VB)],
                vin.at[b],
                isem[b],
            )

        def out_copy(ch, b):
            return pltpu.make_async_copy(
                vout.at[b],
                out_hbm.at[pl.ds((b_lo + CHB * ch) * VB, CHB * VB)],
                osem[b],
            )

        def transpose(b, nv):
            @plsc.parallel_loop(0, LANES * (DIM // 16) * (nv // 16), 1,
                                unroll=8)
            def _(q):
                s = q & 15
                r = q >> 4
                c0 = (r & 3) * LANES
                v0 = (r >> 2) * LANES
                rot = (lane_iota + s) & 15
                v = plsc.load_gather(
                    vin.at[b], [rot + c0, lane_iota + v0]
                )
                plsc.store_scatter(
                    vout.at[b], [lane_iota + v0, rot + c0], v * SCALE
                )

        in_copy(0, 0).start()
        in_copy(1, 1).start()

        @pl.loop(0, nch // 2)
        def outer(t):
            for b in range(2):
                ch = 2 * t + b
                in_copy(ch, b).wait()

                @pl.when(t > 0)
                def _():
                    out_copy(ch - 2, b).wait()

                transpose(b, CHB * VB)
                out_copy(ch, b).start()

                @pl.when(t < nch // 2 - 1)
                def _():
                    in_copy(ch + 2, b).start()

        out_copy(nch - 2, 0).wait()
        out_copy(nch - 1, 1).wait()

        @pl.when(nblk > nch * CHB)
        def _extra():
            vb = b_lo + nch * CHB
            pltpu.make_async_copy(
                tabT_hbm.at[:, pl.ds(vb * VB, VB)],
                vin.at[0].at[:, pl.ds(0, VB)],
                isem[0],
            ).start()
            pltpu.make_async_copy(
                tabT_hbm.at[:, pl.ds(vb * VB, VB)],
                vin.at[0].at[:, pl.ds(0, VB)],
                isem[0],
            ).wait()
            transpose(0, VB)
            pltpu.make_async_copy(
                vout.at[0].at[pl.ds(0, VB)],
                out_hbm.at[pl.ds(vb * VB, VB)],
                osem[0],
            ).start()
            pltpu.make_async_copy(
                vout.at[0].at[pl.ds(0, VB)],
                out_hbm.at[pl.ds(vb * VB, VB)],
                osem[0],
            ).wait()

        @pl.when(wid == NW - 1)
        def _tail():
            pltpu.make_async_copy(tail_hbm, vt, isem[0]).start()
            pltpu.make_async_copy(tail_hbm, vt, isem[0]).wait()

            @plsc.parallel_loop(0, ntail, 1, unroll=4)
            def _(r):
                for h in range(DIM // LANES):
                    sl = pl.ds(LANES * h, LANES)
                    vout[0, r, sl] = vt[r, sl] * SCALE

            pltpu.make_async_copy(
                vout.at[0].at[pl.ds(0, ntail)],
                out_hbm.at[pl.ds(vtail, ntail)],
                osem[0],
            ).start()
            pltpu.make_async_copy(
                vout.at[0].at[pl.ds(0, ntail)],
                out_hbm.at[pl.ds(vtail, ntail)],
                osem[0],
            ).wait()

    return transpose_kernel(tabT, tail)


def kernel(tokens, table):
    B, L = tokens.shape
    BW = B // NW
    CT = DIM // 8
    nt = L // NBUF

    padded = _transpose_table(table)

    idx3 = jnp.transpose(
        tokens.astype(jnp.int32).reshape(NW, BW, L), (0, 2, 1)
    )

    @functools.partial(
        pl.kernel,
        mesh=_mesh(),
        compiler_params=_params,
        out_type=jax.ShapeDtypeStruct((L, CT, NW, 8, LANES * 8), jnp.float32),
        scratch_types=[
            pltpu.VMEM((L, BW), jnp.int32),
            pltpu.VMEM((NBUF, BW, 2 * DIM), jnp.float32),
            pltpu.VMEM((XBUF, DIM, BW), jnp.float32),
        ]
        + [pltpu.SemaphoreType.DMA] * (NBUF + XBUF),
    )
    def emb_kernel(idx_hbm, tab_hbm, out_hbm, tok_v, raw_v, xout_v, *sems):
        gsem = sems[:NBUF]
        osem = sems[NBUF:]
        wid = lax.axis_index("s") * NC + lax.axis_index("c")
        pltpu.sync_copy(idx_hbm.at[wid], tok_v)
        lane_iota = lax.iota(jnp.int32, LANES)

        def gather(c, b):
            return pltpu.make_async_copy(
                tab_hbm.at[tok_v.at[c]], raw_v.at[b], gsem[b]
            )

        def slab_stores(c, xb):
            return [
                pltpu.make_async_copy(
                    xout_v.at[xb].at[pl.ds(ct * 8, 8)],
                    out_hbm.at[c, ct, wid],
                    osem[xb],
                )
                for ct in range(CT)
            ]

        gather(0, 0).start()
        gather(1, 1).start()

        @pl.loop(0, nt)
        def outer(t):
            for b in range(NBUF):
                c = t * NBUF + b
                xb = b % XBUF
                if b < 2:
                    gather(c + 2, (b + 2) % NBUF).start()
                else:
                    @pl.when(t < nt - 1)
                    def _():
                        gather(c + 2, (b + 2) % NBUF).start()

                gather(c, b).wait()

                if b < 2:
                    @pl.when(t > 0)
                    def _():
                        for cp in slab_stores(c - 2, xb):
                            cp.wait()
                else:
                    for cp in slab_stores(c - 2, xb):
                        cp.wait()

                @plsc.parallel_loop(0, LANES * 32, 1, unroll=8)
                def transpose(q):
                    s = q & 15
                    blk = q >> 4
                    c0 = (blk >> 3) * LANES
                    r0 = (blk & 7) * LANES
                    rot = (lane_iota + s) & 15
                    rows = lane_iota + r0
                    cols = rot + c0
                    v = plsc.load_gather(raw_v.at[b], [rows, cols])
                    plsc.store_scatter(xout_v.at[xb], [cols, rows], v)

                for cp in slab_stores(c, xb):
                    cp.start()

        for cc, xb in ((L - 2, 0), (L - 1, 1)):
            for cp in slab_stores(cc, xb):
                cp.wait()

    out5 = emb_kernel(idx3, padded)
    return jnp.transpose(out5, (2, 4, 0, 1, 3)).reshape(B, L, DIM)


# speedup vs baseline: 1.7690x; 1.0462x over previous
import functools

import jax
import jax.numpy as jnp
from jax import lax
from jax.experimental import pallas as pl
from jax.experimental.pallas import tpu as pltpu
from jax.experimental.pallas import tpu_sc as plsc

DIM = 64
SCALE = 8.0
NC, NS, LANES = 2, 16, 16
NW = NC * NS
NBUF = 4
XBUF = 2
VB = 128
CHB = 1

_params = pltpu.CompilerParams(
    use_tc_tiling_on_sc=True, needs_layout_passes=False
)


def _mesh():
    return plsc.VectorSubcoreMesh(core_axis_name="c", subcore_axis_name="s")


def _transpose_table(table):
    V = table.shape[0]
    nfull = V // VB
    vtail = nfull * VB
    ntail = V - vtail
    tabT = jnp.transpose(table)
    tail = table[vtail:]

    @functools.partial(
        pl.kernel,
        mesh=_mesh(),
        compiler_params=_params,
        out_type=jax.ShapeDtypeStruct((V, 2 * DIM), jnp.float32),
        scratch_types=[
            pltpu.VMEM((2, DIM, CHB * VB), jnp.float32),
            pltpu.VMEM((2, CHB * VB, 2 * DIM), jnp.float32),
            pltpu.VMEM((ntail, DIM), jnp.float32),
        ]
        + [pltpu.SemaphoreType.DMA] * 4,
    )
    def transpose_kernel(tabT_hbm, tail_hbm, out_hbm, vin, vout, vt, *sems):
        isem = sems[:2]
        osem = sems[2:]
        wid = lax.axis_index("s") * NC + lax.axis_index("c")
        b_lo = nfull * wid // NW
        nblk = nfull * (wid + 1) // NW - b_lo
        nch = 244 // CHB
        lane_iota = lax.iota(jnp.int32, LANES)

        def in_copy(ch, b):
            return pltpu.make_async_copy(
                tabT_hbm.at[:, pl.ds((b_lo + CHB * ch) * VB, CHB * VB)],
                vin.at[b],
                isem[b],
            )

        def out_copy(ch, b):
            return pltpu.make_async_copy(
                vout.at[b],
                out_hbm.at[pl.ds((b_lo + CHB * ch) * VB, CHB * VB)],
                osem[b],
            )

        def transpose(b, nv):
            @plsc.parallel_loop(0, LANES * (DIM // 16) * (nv // 16), 1,
                                unroll=8)
            def _(q):
                s = q & 15
                r = q >> 4
                c0 = (r & 3) * LANES
                v0 = (r >> 2) * LANES
                rot = (lane_iota + s) & 15
                v = plsc.load_gather(
                    vin.at[b], [rot + c0, lane_iota + v0]
                )
                plsc.store_scatter(
                    vout.at[b], [lane_iota + v0, rot + c0], v * SCALE
                )

        in_copy(0, 0).start()
        in_copy(1, 1).start()

        @pl.loop(0, nch // 2)
        def outer(t):
            for b in range(2):
                ch = 2 * t + b
                in_copy(ch, b).wait()

                @pl.when(t > 0)
                def _():
                    out_copy(ch - 2, b).wait()

                transpose(b, CHB * VB)
                out_copy(ch, b).start()

                @pl.when(t < nch // 2 - 1)
                def _():
                    in_copy(ch + 2, b).start()

        out_copy(nch - 2, 0).wait()
        out_copy(nch - 1, 1).wait()

        @pl.when(nblk > nch * CHB)
        def _extra():
            vb = b_lo + nch * CHB
            pltpu.make_async_copy(
                tabT_hbm.at[:, pl.ds(vb * VB, VB)],
                vin.at[0].at[:, pl.ds(0, VB)],
                isem[0],
            ).start()
            pltpu.make_async_copy(
                tabT_hbm.at[:, pl.ds(vb * VB, VB)],
                vin.at[0].at[:, pl.ds(0, VB)],
                isem[0],
            ).wait()
            transpose(0, VB)
            pltpu.make_async_copy(
                vout.at[0].at[pl.ds(0, VB)],
                out_hbm.at[pl.ds(vb * VB, VB)],
                osem[0],
            ).start()
            pltpu.make_async_copy(
                vout.at[0].at[pl.ds(0, VB)],
                out_hbm.at[pl.ds(vb * VB, VB)],
                osem[0],
            ).wait()

        @pl.when(wid == NW - 1)
        def _tail():
            pltpu.make_async_copy(tail_hbm, vt, isem[0]).start()
            pltpu.make_async_copy(tail_hbm, vt, isem[0]).wait()

            @plsc.parallel_loop(0, ntail, 1, unroll=4)
            def _(r):
                for h in range(DIM // LANES):
                    sl = pl.ds(LANES * h, LANES)
                    vout[0, r, sl] = vt[r, sl] * SCALE

            pltpu.make_async_copy(
                vout.at[0].at[pl.ds(0, ntail)],
                out_hbm.at[pl.ds(vtail, ntail)],
                osem[0],
            ).start()
            pltpu.make_async_copy(
                vout.at[0].at[pl.ds(0, ntail)],
                out_hbm.at[pl.ds(vtail, ntail)],
                osem[0],
            ).wait()

    return transpose_kernel(tabT, tail)


def kernel(tokens, table):
    B, L = tokens.shape
    BW = B // NW
    CT = DIM // 8
    nt = L // NBUF

    padded = _transpose_table(table)

    idx3 = jnp.transpose(
        tokens.astype(jnp.int32).reshape(NW, BW, L), (0, 2, 1)
    )

    @functools.partial(
        pl.kernel,
        mesh=_mesh(),
        compiler_params=_params,
        out_type=jax.ShapeDtypeStruct((L, CT, NW, 8, LANES * 8), jnp.float32),
        scratch_types=[
            pltpu.VMEM((L, BW), jnp.int32),
            pltpu.VMEM((NBUF, BW, 2 * DIM), jnp.float32),
            pltpu.VMEM((XBUF, DIM, BW), jnp.float32),
        ]
        + [pltpu.SemaphoreType.DMA] * (NBUF + XBUF),
    )
    def emb_kernel(idx_hbm, tab_hbm, out_hbm, tok_v, raw_v, xout_v, *sems):
        gsem = sems[:NBUF]
        osem = sems[NBUF:]
        wid = lax.axis_index("s") * NC + lax.axis_index("c")
        pltpu.sync_copy(idx_hbm.at[wid], tok_v)
        lane_iota = lax.iota(jnp.int32, LANES)

        def gather(c, b):
            return pltpu.make_async_copy(
                tab_hbm.at[tok_v.at[c]], raw_v.at[b], gsem[b]
            )

        def slab_stores(c, xb):
            return [
                pltpu.make_async_copy(
                    xout_v.at[xb].at[pl.ds(ct * 8, 8)],
                    out_hbm.at[c, ct, wid],
                    osem[xb],
                )
                for ct in range(CT)
            ]

        gather(0, 0).start()
        gather(1, 1).start()

        @pl.loop(0, nt)
        def outer(t):
            for b in range(NBUF):
                c = t * NBUF + b
                xb = b % XBUF
                if b < 2:
                    gather(c + 2, (b + 2) % NBUF).start()
                else:
                    @pl.when(t < nt - 1)
                    def _():
                        gather(c + 2, (b + 2) % NBUF).start()

                gather(c, b).wait()

                if b < 2:
                    @pl.when(t > 0)
                    def _():
                        for cp in slab_stores(c - 2, xb):
                            cp.wait()
                else:
                    for cp in slab_stores(c - 2, xb):
                        cp.wait()

                @plsc.parallel_loop(0, LANES * 32, 1, unroll=8)
                def transpose(q):
                    s = q & 15
                    blk = q >> 4
                    c0 = (blk >> 3) * LANES
                    r0 = (blk & 7) * LANES
                    rot = (lane_iota + s) & 15
                    rows = lane_iota + r0
                    cols = rot + c0
                    v = plsc.load_gather(raw_v.at[b], [rows, cols])
                    plsc.store_scatter(xout_v.at[xb], [cols, rows], v)

                for cp in slab_stores(c, xb):
                    cp.start()

        for cc, xb in ((L - 2, 0), (L - 1, 1)):
            for cp in slab_stores(cc, xb):
                cp.wait()

    out5 = emb_kernel(idx3, padded)
    return jnp.transpose(out5, (2, 4, 0, 1, 3)).reshape(B, L, DIM)
